# Initial kernel scaffold; baseline (speedup 1.0000x reference)
import jax
import jax.numpy as jnp
from jax.experimental import pallas as pl


def kernel(h, e, kr, edge_index, Wq, bq, Wk, bk, Wv, bv, We, be, Wkr, bkr):
    raise NotImplementedError("write your pallas kernel here")


# speedup vs baseline: 45.4671x; 45.4671x over previous
import functools

import jax
import jax.numpy as jnp
import numpy as np
from jax import lax
from jax.experimental import pallas as pl
from jax.experimental.pallas import tpu as pltpu
from jax.experimental.pallas import tpu_sc as plsc

_N = 10000
_E = 320000
_HD = 128
_H = 8
_D = 16

_ROW_BLK = 1000
_EDGE_BLK = 2000
_GW = 64


def _proj_body(h_ref, wq_ref, bq_ref, wk_ref, bk_ref, wv_ref, bv_ref,
               q_ref, k_ref, v_ref):
    hb = h_ref[...]
    q_ref[...] = jnp.dot(hb, wq_ref[...], preferred_element_type=jnp.float32) + bq_ref[...]
    k_ref[...] = jnp.dot(hb, wk_ref[...], preferred_element_type=jnp.float32) + bk_ref[...]
    v_ref[...] = jnp.dot(hb, wv_ref[...], preferred_element_type=jnp.float32) + bv_ref[...]


def _proj_nodes(h, Wq, bq, Wk, bk, Wv, bv):
    n = h.shape[0]
    grid = n // _ROW_BLK
    row_spec = pl.BlockSpec((_ROW_BLK, _HD), lambda i: (i, 0))
    w_spec = pl.BlockSpec((_HD, _HD), lambda i: (0, 0))
    b_spec = pl.BlockSpec((1, _HD), lambda i: (0, 0))
    out = pl.pallas_call(
        _proj_body,
        grid=(grid,),
        in_specs=[row_spec, w_spec, b_spec, w_spec, b_spec, w_spec, b_spec],
        out_specs=[row_spec, row_spec, row_spec],
        out_shape=[jax.ShapeDtypeStruct((n, _HD), jnp.float32)] * 3,
    )(h, Wq, bq.reshape(1, _HD), Wk, bk.reshape(1, _HD), Wv, bv.reshape(1, _HD))
    return out


def _gather2(tab_a, idx_a2, tab_b, idx_b2):
    mesh = plsc.VectorSubcoreMesh(core_axis_name="core", subcore_axis_name="subcore")

    @functools.partial(
        pl.kernel,
        out_type=[jax.ShapeDtypeStruct((_E, _HD), jnp.float32)] * 2,
        mesh=mesh,
    )
    def k(ta_hbm, tb_hbm, ia_hbm, ib_hbm, oa_hbm, ob_hbm):
        def body(ia_v, ib_v, oa_v, ob_v):
            pltpu.sync_copy(ta_hbm.at[ia_v.at[0, 0]], oa_v)
            pltpu.sync_copy(tb_hbm.at[ib_v.at[0, 0]], ob_v)

        pltpu.emit_pipeline(
            body,
            grid=(_E // _GW,),
            in_specs=[pl.BlockSpec((1, 1, _GW), lambda i: (i, 0, 0)),
                      pl.BlockSpec((1, 1, _GW), lambda i: (i, 0, 0))],
            out_specs=[pl.BlockSpec((_GW, _HD), lambda i: (i, 0)),
                       pl.BlockSpec((_GW, _HD), lambda i: (i, 0))],
            core_axis_name=("core", "subcore"),
            dimension_semantics=(pltpu.PARALLEL,),
        )(ia_hbm, ib_hbm, oa_hbm, ob_hbm)

    return k(tab_a, tab_b, idx_a2, idx_b2)


def _gather1(tab, idx2):
    mesh = plsc.VectorSubcoreMesh(core_axis_name="core", subcore_axis_name="subcore")

    @functools.partial(
        pl.kernel,
        out_type=jax.ShapeDtypeStruct((_E, _HD), jnp.float32),
        mesh=mesh,
    )
    def k(tab_hbm, idx_hbm, out_hbm):
        def body(idx_v, out_v):
            pltpu.sync_copy(tab_hbm.at[idx_v.at[0, 0]], out_v)

        pltpu.emit_pipeline(
            body,
            grid=(_E // _GW,),
            in_specs=[pl.BlockSpec((1, 1, _GW), lambda i: (i, 0, 0))],
            out_specs=[pl.BlockSpec((_GW, _HD), lambda i: (i, 0))],
            core_axis_name=("core", "subcore"),
            dimension_semantics=(pltpu.PARALLEL,),
        )(idx_hbm, out_hbm)

    return k(tab, idx2)


def _edge_body(e_ref, kr_ref, kg_ref, qg_ref, vg_ref, we_ref, be_ref,
               wkr_ref, bkr_ref, bd_ref, c16_ref, eout_ref, m_ref, m1_ref, s_ref):
    pe = jnp.dot(e_ref[...], we_ref[...], preferred_element_type=jnp.float32) + be_ref[...]
    pkr = jnp.dot(kr_ref[...], wkr_ref[...], preferred_element_type=jnp.float32) + bkr_ref[...]
    score = (kg_ref[...] * qg_ref[...] * 0.25 + pkr) * pe
    eout_ref[...] = score
    ssum = jnp.dot(score, bd_ref[...], preferred_element_type=jnp.float32)
    sexp = jnp.exp(jnp.clip(ssum, -5.0, 5.0))
    msg = vg_ref[...] * sexp
    m_ref[...] = msg[:, :64]
    m1_ref[...] = msg[:, 64:]
    s16 = jnp.dot(score, c16_ref[...], preferred_element_type=jnp.float32)
    lane = lax.broadcasted_iota(jnp.int32, s16.shape, 1)
    s_ref[...] = jnp.where(lane < _H, jnp.exp(jnp.clip(s16, -5.0, 5.0)), 0.0)


def _edge_stage(e, kr, Kg, Qg, Vg, We, be, Wkr, bkr, bd, c16):
    grid = _E // _EDGE_BLK
    row_spec = pl.BlockSpec((_EDGE_BLK, _HD), lambda i: (i, 0))
    w_spec = pl.BlockSpec((_HD, _HD), lambda i: (0, 0))
    b_spec = pl.BlockSpec((1, _HD), lambda i: (0, 0))
    c16_spec = pl.BlockSpec((_HD, 16), lambda i: (0, 0))
    s_spec = pl.BlockSpec((_EDGE_BLK, 16), lambda i: (i, 0))
    half_spec = pl.BlockSpec((_EDGE_BLK, 64), lambda i: (i, 0))
    return pl.pallas_call(
        _edge_body,
        grid=(grid,),
        in_specs=[row_spec, row_spec, row_spec, row_spec, row_spec,
                  w_spec, b_spec, w_spec, b_spec, w_spec, c16_spec],
        out_specs=[row_spec, half_spec, half_spec, s_spec],
        out_shape=[jax.ShapeDtypeStruct((_E, _HD), jnp.float32),
                   jax.ShapeDtypeStruct((_E, 64), jnp.float32),
                   jax.ShapeDtypeStruct((_E, 64), jnp.float32),
                   jax.ShapeDtypeStruct((_E, 16), jnp.float32)],
    )(e, kr, Kg, Qg, Vg, We, be.reshape(1, _HD), Wkr, bkr.reshape(1, _HD), bd, c16)


_ROWS_STRIPE = 624


def _zero_stage(stg, ncols):
    @pl.loop(0, 64)
    def _(r):
        @pl.loop(0, ncols, step=16)
        def _(c):
            stg.at[pl.ds(r, 1), pl.ds(c, 16)][...] = jnp.zeros((1, 16), jnp.float32)


def _striped(sid, fn):
    @pl.when(sid < 15)
    def _():
        base = sid * _ROWS_STRIPE
        for off in range(0, _ROWS_STRIPE, 64):
            fn(base + off, min(64, _ROWS_STRIPE - off))

    @pl.when(sid == 15)
    def _():
        base = 15 * _ROWS_STRIPE
        tail = _N - 15 * _ROWS_STRIPE
        for off in range(0, tail, 64):
            fn(base + off, min(64, tail - off))


def _aggregate_wv(m0, m1, dst2):
    mesh = plsc.VectorSubcoreMesh(core_axis_name="core", subcore_axis_name="subcore")
    _HC = _HD // 2

    @functools.partial(
        pl.kernel,
        out_type=jax.ShapeDtypeStruct((2, _N, _HC), jnp.float32),
        mesh=mesh,
        scratch_types=[pltpu.VMEM_SHARED((_N, _HC), jnp.float32),
                       pltpu.VMEM((64, _HC), jnp.float32)],
    )
    def k(m0_hbm, m1_hbm, dst_hbm, wv_out, wv_acc, wv_stg):
        cid = lax.axis_index("core")
        sid = lax.axis_index("subcore")

        _zero_stage(wv_stg, _HC)

        def init_stripe(base, nrows):
            pltpu.sync_copy(wv_stg.at[pl.ds(0, nrows)], wv_acc.at[pl.ds(base, nrows)])

        _striped(sid, init_stripe)
        plsc.subcore_barrier()

        def body(m0_v, m1_v, idx_v):
            @pl.when(cid == 0)
            def _():
                pltpu.sync_copy(m0_v, wv_acc.at[idx_v.at[0, 0]], add=True)

            @pl.when(cid == 1)
            def _():
                pltpu.sync_copy(m1_v, wv_acc.at[idx_v.at[0, 0]], add=True)

        pltpu.emit_pipeline(
            body,
            grid=(_E // _GW,),
            in_specs=[pl.BlockSpec((_GW, _HC), lambda i: (i, 0)),
                      pl.BlockSpec((_GW, _HC), lambda i: (i, 0)),
                      pl.BlockSpec((1, 1, _GW), lambda i: (i, 0, 0))],
            out_specs=[],
            core_axis_name="subcore",
            dimension_semantics=(pltpu.PARALLEL,),
        )(m0_hbm, m1_hbm, dst_hbm)

        plsc.subcore_barrier()

        def drain_stripe(base, nrows):
            pltpu.sync_copy(wv_acc.at[pl.ds(base, nrows)], wv_stg.at[pl.ds(0, nrows)])
            pltpu.sync_copy(wv_stg.at[pl.ds(0, nrows)],
                            wv_out.at[cid, pl.ds(base, nrows)])

        _striped(sid, drain_stripe)

    return k(m0, m1, dst2)


def _aggregate_z(s16, dst2):
    mesh = plsc.VectorSubcoreMesh(core_axis_name="core", subcore_axis_name="subcore")

    @functools.partial(
        pl.kernel,
        out_type=jax.ShapeDtypeStruct((2, _N, 16), jnp.float32),
        mesh=mesh,
        scratch_types=[pltpu.VMEM_SHARED((_N, 16), jnp.float32),
                       pltpu.VMEM((64, 16), jnp.float32)],
    )
    def k(s_hbm, dst_hbm, z_out, z_acc, z_stg):
        cid = lax.axis_index("core")
        sid = lax.axis_index("subcore")

        _zero_stage(z_stg, 16)

        def init_stripe(base, nrows):
            pltpu.sync_copy(z_stg.at[pl.ds(0, nrows)], z_acc.at[pl.ds(base, nrows)])

        _striped(sid, init_stripe)
        plsc.subcore_barrier()

        def body(s_v, idx_v):
            pltpu.sync_copy(s_v, z_acc.at[idx_v.at[0, 0]], add=True)

        pltpu.emit_pipeline(
            body,
            grid=(_E // _GW,),
            in_specs=[pl.BlockSpec((_GW, 16), lambda i: (i, 0)),
                      pl.BlockSpec((1, 1, _GW), lambda i: (i, 0, 0))],
            out_specs=[],
            core_axis_name=("core", "subcore"),
            dimension_semantics=(pltpu.PARALLEL,),
        )(s_hbm, dst_hbm)

        plsc.subcore_barrier()

        def drain_stripe(base, nrows):
            pltpu.sync_copy(z_acc.at[pl.ds(base, nrows)], z_stg.at[pl.ds(0, nrows)])
            pltpu.sync_copy(z_stg.at[pl.ds(0, nrows)], z_out.at[cid, pl.ds(base, nrows)])

        _striped(sid, drain_stripe)

    return k(s16, dst2)


def _final_body(wv_ref, z_ref, e16_ref, out_ref):
    wv = jnp.concatenate([wv_ref[0], wv_ref[1]], axis=1)
    z = z_ref[0] + z_ref[1]
    zb = jnp.dot(z, e16_ref[...], preferred_element_type=jnp.float32)
    out_ref[...] = wv / (zb + 1e-6)


def _finalize(wv_p, z_p, e16):
    grid = _N // _ROW_BLK
    return pl.pallas_call(
        _final_body,
        grid=(grid,),
        in_specs=[pl.BlockSpec((2, _ROW_BLK, 64), lambda i: (0, i, 0)),
                  pl.BlockSpec((2, _ROW_BLK, 16), lambda i: (0, i, 0)),
                  pl.BlockSpec((16, _HD), lambda i: (0, 0))],
        out_specs=pl.BlockSpec((_ROW_BLK, _HD), lambda i: (i, 0)),
        out_shape=jax.ShapeDtypeStruct((_N, _HD), jnp.float32),
    )(wv_p, z_p, e16)


def kernel(h, e, kr, edge_index, Wq, bq, Wk, bk, Wv, bv, We, be, Wkr, bkr):
    src2 = edge_index[0].reshape(_E // _GW, 1, _GW)
    dst2 = edge_index[1].reshape(_E // _GW, 1, _GW)

    hd_i = np.arange(_HD)
    bd = jnp.asarray((hd_i[:, None] // _D) == (hd_i[None, :] // _D), jnp.float32)
    c16 = jnp.asarray((hd_i[:, None] // _D) == np.arange(16)[None, :], jnp.float32)
    e16 = jnp.asarray(np.arange(16)[:, None] == (hd_i[None, :] // _D), jnp.float32)

    Q_h, K_h, V_h = _proj_nodes(h, Wq, bq, Wk, bk, Wv, bv)
    Kg, Qg = _gather2(K_h, src2, Q_h, dst2)
    Vg = _gather1(V_h, src2)
    e_out, m0, m1, s16 = _edge_stage(e, kr, Kg, Qg, Vg, We, be, Wkr, bkr, bd, c16)
    wv_p = _aggregate_wv(m0, m1, dst2)
    z_p = _aggregate_z(s16, dst2)
    h_out = _finalize(wv_p, z_p, e16)
    return h_out.reshape(_N, _H, _D), e_out.reshape(_E, _H, _D)
